# baseline (speedup 1.0000x reference)
import functools

import jax
import jax.numpy as jnp
from jax import lax
from jax.experimental import pallas as pl
from jax.experimental.pallas import tpu as pltpu
from jax.experimental.pallas import tpu_sc as plsc

D = 512
L = 16
S = 50
S0 = 48
G = 224
NBUF = 2


def _sc_add_emb(x, idx4, table4):
    B = x.shape[0]
    info = plsc.get_sparse_core_info()
    NC, NS = info.num_cores, info.num_subcores
    NW = NC * NS
    n_wb = B // NW
    mesh = plsc.VectorSubcoreMesh(core_axis_name="c", subcore_axis_name="s")

    @functools.partial(
        pl.kernel,
        mesh=mesh,
        out_type=jax.ShapeDtypeStruct((B, S, D), jnp.float32),
        scratch_types=[
            pltpu.VMEM((G,), jnp.int32),
            pltpu.VMEM((G,), jnp.int32),
            pltpu.VMEM((NBUF, S, D), jnp.float32),
            pltpu.VMEM((NBUF, G, 128), jnp.float32),
            pltpu.SemaphoreType.DMA((NBUF,)),
            pltpu.SemaphoreType.DMA((NBUF,)),
            pltpu.SemaphoreType.DMA((NBUF,)),
            pltpu.SemaphoreType.DMA((NBUF,)),
        ],
    )
    def k(x_hbm, idx_hbm, tab_hbm, out_hbm,
          idx_v0, idx_v1, x_v, rows_v, sem_i, sem_x, sem_g, sem_o):
        wid = lax.axis_index("s") * NC + lax.axis_index("c")
        wb = wid * n_wb
        idx_bufs = (idx_v0, idx_v1)

        def idx_copy(g, b):
            base = pl.multiple_of((wb + g) * G, G)
            return pltpu.make_async_copy(
                idx_hbm.at[pl.ds(base, G)], idx_bufs[b], sem_i.at[b])

        def x_copies(g, b):
            return (
                pltpu.make_async_copy(
                    x_hbm.at[wb + g, pl.ds(0, S0)],
                    x_v.at[b, pl.ds(0, S0)], sem_x.at[b]),
                pltpu.make_async_copy(
                    x_hbm.at[wb + g, pl.ds(S0, S - S0)],
                    x_v.at[b, pl.ds(S0, S - S0)], sem_x.at[b]),
            )

        def gather_copies(g, b):
            return (
                pltpu.make_async_copy(
                    tab_hbm.at[idx_bufs[b].at[pl.ds(0, G // 2)]],
                    rows_v.at[b, pl.ds(0, G // 2)], sem_g.at[b]),
                pltpu.make_async_copy(
                    tab_hbm.at[idx_bufs[b].at[pl.ds(G // 2, G // 2)]],
                    rows_v.at[b, pl.ds(G // 2, G // 2)], sem_g.at[b]),
            )

        def out_copies(g, b):
            return (
                pltpu.make_async_copy(
                    x_v.at[b, pl.ds(0, S0)],
                    out_hbm.at[wb + g, pl.ds(0, S0)], sem_o.at[b]),
                pltpu.make_async_copy(
                    x_v.at[b, pl.ds(S0, S - S0)],
                    out_hbm.at[wb + g, pl.ds(S0, S - S0)], sem_o.at[b]),
            )

        def issue_in(g, b):
            idx_copy(g, b).start()
            for cp in x_copies(g, b):
                cp.start()
            idx_copy(g, b).wait()
            for cp in gather_copies(g, b):
                cp.start()

        def step(g, b):
            for cp in x_copies(g, b):
                cp.wait()
            for cp in gather_copies(g, b):
                cp.wait()

            def row(r, carry2):
                for lt in range(4):
                    for s in range(8):
                        plsc.addupdate(
                            x_v.at[b, r, pl.ds(lt * 128 + s * L, L)],
                            rows_v[b, 4 * r + lt, pl.ds(s * L, L)],
                        )
                return carry2

            lax.fori_loop(0, S, row, 0)
            for cp in out_copies(g, b):
                cp.start()

            g1 = g + 1
            b1 = (b + 1) % NBUF

            @pl.when(jnp.logical_and(g >= 1, g1 < n_wb))
            def _():
                for cp in out_copies(g - 1, b1):
                    cp.wait()

            @pl.when(g1 < n_wb)
            def _():
                issue_in(g1, b1)

        issue_in(0, 0)

        def body(gg, carry):
            g0 = gg * NBUF
            for par in range(NBUF):
                step(g0 + par, par)
            return carry

        lax.fori_loop(0, n_wb // NBUF, body, 0)

        for gd in range(n_wb - NBUF, n_wb):
            for cp in out_copies(gd, gd % NBUF):
                cp.wait()

    return k(x, idx4, table4)


def kernel(x, instruction_types, type_emb_weight):
    idx = instruction_types.astype(jnp.int32)
    idx4 = (idx[:, :, None] * 4 + jnp.arange(4, dtype=jnp.int32)).reshape(
        idx.shape[0], 4 * S)
    idx4 = jnp.pad(idx4, ((0, 0), (0, G - 4 * S))).reshape(-1)
    table4 = type_emb_weight.reshape(4000, 128)
    return _sc_add_emb(x, idx4, table4)


# speedup vs baseline: 1.5081x; 1.5081x over previous
import functools

import jax
import jax.numpy as jnp
from jax import lax
from jax.experimental import pallas as pl
from jax.experimental.pallas import tpu as pltpu
from jax.experimental.pallas import tpu_sc as plsc

D = 512
DH = 256
L = 16
S = 50
SP = 56
NBUF = 4
PF = 2


def _sc_add_emb(x, idx2, table2):
    B = x.shape[0]
    info = plsc.get_sparse_core_info()
    NC, NS = info.num_cores, info.num_subcores
    NW = NC * NS
    n_c = 2 * B // NW
    mesh = plsc.VectorSubcoreMesh(core_axis_name="c", subcore_axis_name="s")

    @functools.partial(
        pl.kernel,
        mesh=mesh,
        out_type=jax.ShapeDtypeStruct((B, S, D), jnp.float32),
        scratch_types=[
            pltpu.VMEM((SP,), jnp.int32),
            pltpu.VMEM((SP,), jnp.int32),
            pltpu.VMEM((SP,), jnp.int32),
            pltpu.VMEM((SP,), jnp.int32),
            pltpu.VMEM((NBUF, S, DH), jnp.float32),
            pltpu.VMEM((NBUF, SP, DH), jnp.float32),
            pltpu.SemaphoreType.DMA((NBUF,)),
            pltpu.SemaphoreType.DMA((NBUF,)),
            pltpu.SemaphoreType.DMA((NBUF,)),
            pltpu.SemaphoreType.DMA((NBUF,)),
        ],
    )
    def k(x_hbm, idx_hbm, tab_hbm, out_hbm,
          iv0, iv1, iv2, iv3, x_v, rows_v, sem_i, sem_x, sem_g, sem_o):
        wid = lax.axis_index("s") * NC + lax.axis_index("c")
        wc = wid * n_c
        idx_bufs = (iv0, iv1, iv2, iv3)

        def batch_of(g):
            return wid * (n_c // 2) + lax.div(g, 2)

        def idx_copy(g, b):
            base = pl.multiple_of((wc + g) * SP, SP)
            return pltpu.make_async_copy(
                idx_hbm.at[pl.ds(base, SP)], idx_bufs[b], sem_i.at[b])

        def x_copy(g, b, h):
            return pltpu.make_async_copy(
                x_hbm.at[batch_of(g), :, pl.ds(h * DH, DH)],
                x_v.at[b], sem_x.at[b])

        def gather_copy(g, b):
            return pltpu.make_async_copy(
                tab_hbm.at[idx_bufs[b]], rows_v.at[b], sem_g.at[b])

        def out_copy(g, b, h):
            return pltpu.make_async_copy(
                x_v.at[b], out_hbm.at[batch_of(g), :, pl.ds(h * DH, DH)],
                sem_o.at[b])

        for g0 in range(PF + 1):
            idx_copy(g0, g0 % NBUF).start()
        for g0 in range(PF):
            x_copy(g0, g0 % NBUF, g0 % 2).start()
            idx_copy(g0, g0 % NBUF).wait()
            gather_copy(g0, g0 % NBUF).start()

        def step(g, b, h):
            x_copy(g, b, h).wait()
            gather_copy(g, b).wait()

            def row(r, carry2):
                for j in range(DH // L):
                    plsc.addupdate(
                        x_v.at[b, r, pl.ds(j * L, L)],
                        rows_v[b, r, pl.ds(j * L, L)],
                    )
                return carry2

            lax.fori_loop(0, S, row, 0)
            out_copy(g, b, h).start()

            g2 = g + PF
            b2 = (b + PF) % NBUF

            @pl.when(jnp.logical_and(g >= PF, g2 < n_c))
            def _():
                out_copy(g - PF, b2, h).wait()

            @pl.when(g2 < n_c)
            def _():
                idx_copy(g2, b2).wait()
                gather_copy(g2, b2).start()
                x_copy(g2, b2, h).start()

            g3 = g + PF + 1
            b3 = (b + PF + 1) % NBUF

            @pl.when(g3 < n_c)
            def _():
                idx_copy(g3, b3).start()

        def body(gg, carry):
            g0 = gg * NBUF
            for par in range(NBUF):
                step(g0 + par, par, par % 2)
            return carry

        lax.fori_loop(0, n_c // NBUF, body, 0)

        for gd in range(n_c - NBUF, n_c):
            out_copy(gd, gd % NBUF, gd % 2).wait()

    return k(x, idx2, table2)


def kernel(x, instruction_types, type_emb_weight):
    idx = instruction_types.astype(jnp.int32)
    idx2 = 2 * idx[:, None, :] + jnp.arange(2, dtype=jnp.int32)[None, :, None]
    idx2 = jnp.pad(idx2, ((0, 0), (0, 0), (0, SP - S))).reshape(-1)
    table2 = type_emb_weight.reshape(2000, DH)
    return _sc_add_emb(x, idx2, table2)


# speedup vs baseline: 3.2255x; 2.1388x over previous
import functools

import jax
import jax.numpy as jnp
from jax import lax
from jax.experimental import pallas as pl
from jax.experimental.pallas import tpu as pltpu
from jax.experimental.pallas import tpu_sc as plsc

D = 512
S = 50
C = 32
NBUF = 3
BB = 4


def _sc_gather(table, idx):
    N = idx.shape[0]
    info = plsc.get_sparse_core_info()
    NC, NS = info.num_cores, info.num_subcores
    NW = NC * NS
    n_w = N // NW
    n_chunks = n_w // C
    mesh = plsc.VectorSubcoreMesh(core_axis_name="c", subcore_axis_name="s")

    @functools.partial(
        pl.kernel,
        mesh=mesh,
        out_type=jax.ShapeDtypeStruct((N, D), jnp.float32),
        scratch_types=[
            pltpu.VMEM((n_w,), jnp.int32),
            pltpu.VMEM((NBUF, C, D), jnp.float32),
            pltpu.SemaphoreType.DMA((NBUF,)),
            pltpu.SemaphoreType.DMA((NBUF,)),
        ],
    )
    def k(tab_hbm, idx_hbm, out_hbm, idx_all, rows_v, sem_g, sem_o):
        wid = lax.axis_index("s") * NC + lax.axis_index("c")
        wbase = wid * n_w
        pltpu.sync_copy(idx_hbm.at[pl.ds(wbase, n_w)], idx_all)

        def gather_copy(g, b):
            ibase = pl.multiple_of(g * C, C)
            return pltpu.make_async_copy(
                tab_hbm.at[idx_all.at[pl.ds(ibase, C)]],
                rows_v.at[b], sem_g.at[b])

        def out_copy(g, b):
            base = pl.multiple_of(wbase + g * C, C)
            return pltpu.make_async_copy(
                rows_v.at[b], out_hbm.at[pl.ds(base, C)], sem_o.at[b])

        def issue_in(g):
            gather_copy(g, lax.rem(g, NBUF)).start()

        for g0 in range(NBUF - 1):
            issue_in(g0)

        def body(g, carry):
            b = lax.rem(g, NBUF)
            gather_copy(g, b).wait()

            @pl.when(g >= NBUF)
            def _():
                out_copy(g - NBUF, b).wait()

            out_copy(g, b).start()

            g2 = g + NBUF - 1

            @pl.when(g2 < n_chunks)
            def _():
                issue_in(g2)

            return carry

        lax.fori_loop(0, n_chunks, body, 0)

        for gd in range(n_chunks - NBUF, n_chunks):
            out_copy(gd, gd % NBUF).wait()

    return k(table, idx)


def _tc_add_kernel(x_ref, emb_ref, o_ref):
    for kk in range(BB):
        o_ref[kk] = x_ref[kk] + emb_ref[pl.ds(S * kk, S), :]


def _tc_add(x, emb):
    B = x.shape[0]
    grid = (B // BB,)
    return pl.pallas_call(
        _tc_add_kernel,
        grid=grid,
        in_specs=[
            pl.BlockSpec((BB, S, D), lambda i: (i, 0, 0)),
            pl.BlockSpec((BB * S, D), lambda i: (i, 0)),
        ],
        out_specs=pl.BlockSpec((BB, S, D), lambda i: (i, 0, 0)),
        out_shape=jax.ShapeDtypeStruct((B, S, D), jnp.float32),
    )(x, emb)


def kernel(x, instruction_types, type_emb_weight):
    idx = instruction_types.reshape(-1).astype(jnp.int32)
    emb = _sc_gather(type_emb_weight, idx)
    return _tc_add(x, emb)
